# baseline (speedup 1.0000x reference)
import jax
import jax.numpy as jnp
from jax.experimental import pallas as pl
from jax.experimental.pallas import tpu as pltpu

D_MODEL = 768
NUM_EXPERTS = 64
D_EXPERT = 64
TOKENS = 2048
EPB = 8
GRID = NUM_EXPERTS // EPB


def _fused_kernel(x_ref, wg_ref, w1_ref, w2_ref, out_ref, aux_ref, cw_ref):
    g = pl.program_id(0)

    @pl.when(g == 0)
    def _router():
        x = x_ref[...]
        logits = jnp.dot(x, wg_ref[...], preferred_element_type=jnp.float32)
        m = jnp.max(logits, axis=-1, keepdims=True)
        ex = jnp.exp(logits - m)
        probs = ex / jnp.sum(ex, axis=-1, keepdims=True)
        ei = jax.lax.broadcasted_iota(jnp.int32, (TOKENS, NUM_EXPERTS), 1)
        v1 = jnp.max(probs, axis=-1, keepdims=True)
        i1 = jnp.min(jnp.where(probs == v1, ei, NUM_EXPERTS), axis=-1,
                     keepdims=True)
        oh1 = ei == i1
        probs2 = jnp.where(oh1, -jnp.inf, probs)
        v2 = jnp.max(probs2, axis=-1, keepdims=True)
        i2 = jnp.min(jnp.where(probs2 == v2, ei, NUM_EXPERTS), axis=-1,
                     keepdims=True)
        oh2 = ei == i2
        denom = v1 + v2
        cw_ref[...] = (jnp.where(oh1, v1, 0.0) + jnp.where(oh2, v2, 0.0)) / denom
        disp = oh1.astype(jnp.float32) + oh2.astype(jnp.float32)
        fd = jnp.sum(disp, axis=0, keepdims=True)
        pm = jnp.sum(probs, axis=0, keepdims=True)
        aux = (NUM_EXPERTS / (TOKENS * TOKENS)) * jnp.sum(fd * pm)
        aux_ref[...] = jnp.broadcast_to(aux, (1, 1))

    x = x_ref[...]
    w1cat = jnp.concatenate(
        [w1_ref[g * EPB + e] for e in range(EPB)], axis=1)
    h = jnp.dot(x, w1cat, preferred_element_type=jnp.float32)
    h = 0.5 * h * (1.0 + jax.lax.erf(h * 0.7071067811865476))
    er = jax.lax.broadcasted_iota(jnp.int32, (NUM_EXPERTS, EPB * D_EXPERT), 0)
    ec = jax.lax.broadcasted_iota(jnp.int32, (NUM_EXPERTS, EPB * D_EXPERT), 1)
    sel = (er == g * EPB + ec // D_EXPERT).astype(jnp.float32)
    scale = jnp.dot(cw_ref[...], sel, preferred_element_type=jnp.float32)
    h = h * scale
    contrib = jnp.dot(h, w2_ref[...], preferred_element_type=jnp.float32)

    @pl.when(g == 0)
    def _():
        out_ref[...] = contrib

    @pl.when(g > 0)
    def _():
        out_ref[...] += contrib


def kernel(x, W_gate, expert_w1, expert_w2):
    x2d = x.reshape(TOKENS, D_MODEL)
    w2_all = expert_w2.reshape(NUM_EXPERTS * D_EXPERT, D_MODEL)

    out2d, aux = pl.pallas_call(
        _fused_kernel,
        grid=(GRID,),
        in_specs=[
            pl.BlockSpec((TOKENS, D_MODEL), lambda g: (0, 0)),
            pl.BlockSpec((D_MODEL, NUM_EXPERTS), lambda g: (0, 0)),
            pl.BlockSpec((NUM_EXPERTS, D_MODEL, D_EXPERT), lambda g: (0, 0, 0)),
            pl.BlockSpec((EPB * D_EXPERT, D_MODEL), lambda g: (g, 0)),
        ],
        out_specs=(
            pl.BlockSpec((TOKENS, D_MODEL), lambda g: (0, 0)),
            pl.BlockSpec((1, 1), lambda g: (0, 0)),
        ),
        out_shape=(
            jax.ShapeDtypeStruct((TOKENS, D_MODEL), jnp.float32),
            jax.ShapeDtypeStruct((1, 1), jnp.float32),
        ),
        scratch_shapes=[pltpu.VMEM((TOKENS, NUM_EXPERTS), jnp.float32)],
    )(x2d, W_gate, expert_w1, w2_all)

    return out2d.reshape(x.shape), aux[0, 0]


# speedup vs baseline: 1.2083x; 1.2083x over previous
import jax
import jax.numpy as jnp
from jax.experimental import pallas as pl
from jax.experimental.pallas import tpu as pltpu

D_MODEL = 768
NUM_EXPERTS = 64
D_EXPERT = 64
TOKENS = 2048
EPB = 16
GRID = NUM_EXPERTS // EPB


def _fused_kernel(x_ref, wg_ref, w1_ref, w2_ref, out_ref, aux_ref, cw_ref):
    g = pl.program_id(0)

    @pl.when(g == 0)
    def _router():
        x = x_ref[...]
        logits = jnp.dot(x, wg_ref[...], preferred_element_type=jnp.float32)
        m = jnp.max(logits, axis=-1, keepdims=True)
        ex = jnp.exp(logits - m)
        probs = ex / jnp.sum(ex, axis=-1, keepdims=True)
        ei = jax.lax.broadcasted_iota(jnp.int32, (TOKENS, NUM_EXPERTS), 1)
        v1 = jnp.max(probs, axis=-1, keepdims=True)
        i1 = jnp.min(jnp.where(probs == v1, ei, NUM_EXPERTS), axis=-1,
                     keepdims=True)
        oh1 = ei == i1
        probs2 = jnp.where(oh1, -jnp.inf, probs)
        v2 = jnp.max(probs2, axis=-1, keepdims=True)
        i2 = jnp.min(jnp.where(probs2 == v2, ei, NUM_EXPERTS), axis=-1,
                     keepdims=True)
        oh2 = ei == i2
        denom = v1 + v2
        cw_ref[...] = (jnp.where(oh1, v1, 0.0) + jnp.where(oh2, v2, 0.0)) / denom
        disp = oh1.astype(jnp.float32) + oh2.astype(jnp.float32)
        fd = jnp.sum(disp, axis=0, keepdims=True)
        pm = jnp.sum(probs, axis=0, keepdims=True)
        aux = (NUM_EXPERTS / (TOKENS * TOKENS)) * jnp.sum(fd * pm)
        aux_ref[...] = jnp.broadcast_to(aux, (1, 1))

    x = x_ref[...]
    h = jnp.dot(x, w1_ref[...], preferred_element_type=jnp.float32)
    h = 0.5 * h * (1.0 + jax.lax.erf(h * 0.7071067811865476))
    er = jax.lax.broadcasted_iota(jnp.int32, (NUM_EXPERTS, EPB * D_EXPERT), 0)
    ec = jax.lax.broadcasted_iota(jnp.int32, (NUM_EXPERTS, EPB * D_EXPERT), 1)
    sel = (er == g * EPB + ec // D_EXPERT).astype(jnp.float32)
    scale = jnp.dot(cw_ref[...], sel, preferred_element_type=jnp.float32)
    h = h * scale
    contrib = jnp.dot(h, w2_ref[...], preferred_element_type=jnp.float32)

    @pl.when(g == 0)
    def _():
        out_ref[...] = contrib

    @pl.when(g > 0)
    def _():
        out_ref[...] += contrib


def kernel(x, W_gate, expert_w1, expert_w2):
    x2d = x.reshape(TOKENS, D_MODEL)
    w1_all = expert_w1.transpose(1, 0, 2).reshape(D_MODEL, NUM_EXPERTS * D_EXPERT)
    w2_all = expert_w2.reshape(NUM_EXPERTS * D_EXPERT, D_MODEL)

    out2d, aux = pl.pallas_call(
        _fused_kernel,
        grid=(GRID,),
        in_specs=[
            pl.BlockSpec((TOKENS, D_MODEL), lambda g: (0, 0)),
            pl.BlockSpec((D_MODEL, NUM_EXPERTS), lambda g: (0, 0)),
            pl.BlockSpec((D_MODEL, EPB * D_EXPERT), lambda g: (0, g)),
            pl.BlockSpec((EPB * D_EXPERT, D_MODEL), lambda g: (g, 0)),
        ],
        out_specs=(
            pl.BlockSpec((TOKENS, D_MODEL), lambda g: (0, 0)),
            pl.BlockSpec((1, 1), lambda g: (0, 0)),
        ),
        out_shape=(
            jax.ShapeDtypeStruct((TOKENS, D_MODEL), jnp.float32),
            jax.ShapeDtypeStruct((1, 1), jnp.float32),
        ),
        scratch_shapes=[pltpu.VMEM((TOKENS, NUM_EXPERTS), jnp.float32)],
    )(x2d, W_gate, w1_all, w2_all)

    return out2d.reshape(x.shape), aux[0, 0]
